# baseline (speedup 1.0000x reference)
import jax
import jax.numpy as jnp
from jax.experimental import pallas as pl
from jax.experimental.pallas import tpu as pltpu

_BLOCK_ROWS = 400


def _xw_kernel(x_ref, w_ref, o_ref):
    o_ref[...] = jnp.dot(x_ref[...], w_ref[...],
                         preferred_element_type=jnp.float32)


def _layer1_kernel(adj_ref, s1_ref, b1_ref, w10_ref, o_ref):
    h = jnp.dot(adj_ref[...], s1_ref[...],
                preferred_element_type=jnp.float32)
    h = jnp.maximum(h + b1_ref[...], 0.0)
    o_ref[...] = jnp.dot(h, w10_ref[...],
                         preferred_element_type=jnp.float32)


def _layer2_kernel(adj_ref, s2_ref, b10_ref, o_ref):
    o = jnp.dot(adj_ref[...], s2_ref[...],
                preferred_element_type=jnp.float32) + b10_ref[...]
    m = jnp.max(o, axis=1, keepdims=True)
    e = jnp.exp(o - m)
    lse = jnp.log(jnp.sum(e, axis=1, keepdims=True)) + m
    o_ref[...] = o - lse


@jax.jit
def kernel(x, adj, W1, b1, W10, b10):
    n, _ = x.shape
    nhid = W1.shape[1]
    nclass = W10.shape[1]
    nb = n // _BLOCK_ROWS

    b1r = b1.reshape(1, nhid)
    b10r = b10.reshape(1, nclass)

    s1 = pl.pallas_call(
        _xw_kernel,
        out_shape=jax.ShapeDtypeStruct((n, nhid), jnp.float32),
    )(x, W1)

    row_spec = pl.BlockSpec((_BLOCK_ROWS, n), lambda i: (i, 0))

    def full(shape):
        return pl.BlockSpec(shape, lambda i: (0, 0))

    s2 = pl.pallas_call(
        _layer1_kernel,
        grid=(nb,),
        in_specs=[
            row_spec,
            full((n, nhid)),
            full((1, nhid)),
            full((nhid, nclass)),
        ],
        out_specs=pl.BlockSpec((_BLOCK_ROWS, nclass), lambda i: (i, 0)),
        out_shape=jax.ShapeDtypeStruct((n, nclass), jnp.float32),
        compiler_params=pltpu.CompilerParams(
            dimension_semantics=("parallel",)),
    )(adj, s1, b1r, W10)

    out = pl.pallas_call(
        _layer2_kernel,
        grid=(nb,),
        in_specs=[
            row_spec,
            full((n, nclass)),
            full((1, nclass)),
        ],
        out_specs=pl.BlockSpec((_BLOCK_ROWS, nclass), lambda i: (i, 0)),
        out_shape=jax.ShapeDtypeStruct((n, nclass), jnp.float32),
        compiler_params=pltpu.CompilerParams(
            dimension_semantics=("parallel",)),
    )(adj, s2, b10r)

    return out


# speedup vs baseline: 1.0511x; 1.0511x over previous
import jax
import jax.numpy as jnp
from jax import lax
from jax.experimental import pallas as pl
from jax.experimental.pallas import tpu as pltpu

_BLOCK_ROWS = 400


def _gcn_kernel(nb, adj_ref, x_ref, w1_ref, b1_ref, w10_ref, b10_ref,
                out_ref, s1_ref, s2_ref):
    g = pl.program_id(0)

    @pl.when(g == 0)
    def _():
        s1_ref[...] = jnp.dot(x_ref[...], w1_ref[...],
                              preferred_element_type=jnp.float32)

    @pl.when(g < nb)
    def _():
        h = jnp.dot(adj_ref[...], s1_ref[...],
                    preferred_element_type=jnp.float32)
        h = jnp.maximum(h + b1_ref[...], 0.0)
        base = pl.multiple_of(g * _BLOCK_ROWS, _BLOCK_ROWS)
        s2_ref[pl.ds(base, _BLOCK_ROWS), :] = jnp.dot(
            h, w10_ref[...], preferred_element_type=jnp.float32)

    @pl.when(g >= nb)
    def _():
        o = jnp.dot(adj_ref[...], s2_ref[...],
                    preferred_element_type=jnp.float32) + b10_ref[...]
        m = jnp.max(o, axis=1, keepdims=True)
        lse = jnp.log(jnp.sum(jnp.exp(o - m), axis=1, keepdims=True)) + m
        out_ref[...] = o - lse


@jax.jit
def kernel(x, adj, W1, b1, W10, b10):
    n, nfeat = x.shape
    nhid = W1.shape[1]
    nclass = W10.shape[1]
    nb = n // _BLOCK_ROWS

    def body(*refs):
        _gcn_kernel(nb, *refs)

    def const(shape):
        return pl.BlockSpec(shape, lambda g: (0, 0))

    out = pl.pallas_call(
        body,
        grid=(2 * nb,),
        in_specs=[
            pl.BlockSpec((_BLOCK_ROWS, n), lambda g: (lax.rem(g, nb), 0)),
            const((n, nfeat)),
            const((nfeat, nhid)),
            const((1, nhid)),
            const((nhid, nclass)),
            const((1, nclass)),
        ],
        out_specs=pl.BlockSpec((_BLOCK_ROWS, nclass),
                               lambda g: (lax.rem(g, nb), 0)),
        out_shape=jax.ShapeDtypeStruct((n, nclass), jnp.float32),
        scratch_shapes=[
            pltpu.VMEM((n, nhid), jnp.float32),
            pltpu.VMEM((n, nclass), jnp.float32),
        ],
        compiler_params=pltpu.CompilerParams(
            dimension_semantics=("arbitrary",)),
    )(adj, x, W1, b1.reshape(1, nhid), W10, b10.reshape(1, nclass))

    return out
